# Initial kernel scaffold; baseline (speedup 1.0000x reference)
import jax
import jax.numpy as jnp
from jax.experimental import pallas as pl


def kernel(wid, edge_index, root_ids, embedding, W_z, b_z, W_r, U_r, b_ur, W_h, b_h, W_g, b_g):
    raise NotImplementedError("write your pallas kernel here")


# speedup vs baseline: 2.8456x; 2.8456x over previous
import functools
import jax
import jax.numpy as jnp
from jax import lax
from jax.experimental import pallas as pl
from jax.experimental.pallas import tpu as pltpu
from jax.experimental.pallas import tpu_sc as plsc

N = 10000
E = 160000
H = 128
NC = 2
NS = 16
NW = NC * NS
CH = 128
NCHUNK = E // CH
ITER_J = -(-NCHUNK // NS)
NT = 10240
ZROWS = NT // NS
NPAD = 10240
XCH = NPAD // CH
BE = 2000
GE = E // BE
BN = 1000
GN = N // BN
B = 256

def _f32(*shape):
    return jax.ShapeDtypeStruct(shape, jnp.float32)


def _sc_kernel(out_type, scratch_types):
    def deco(body):
        @functools.cache
        def build():
            mesh = plsc.VectorSubcoreMesh(
                core_axis_name="c", subcore_axis_name="s",
                num_cores=NC, num_subcores=NS)
            return pl.kernel(body, out_type=out_type, mesh=mesh,
                             scratch_types=scratch_types)

        def call(*args):
            return build()(*args)

        return call

    return deco


@_sc_kernel(
    out_type=_f32(NPAD, H),
    scratch_types=[pltpu.VMEM((CH,), jnp.int32),
                   pltpu.VMEM((CH, H), jnp.float32)],
)
def _sc_gather_x(emb, wid2d, xpad, idx_v, rows_v):
    w = lax.axis_index("s") * NC + lax.axis_index("c")

    @pl.loop(0, -(-XCH // NW))
    def _(j):
        c = w + NW * j

        @pl.when(c < XCH)
        def _():
            pltpu.sync_copy(wid2d.at[c], idx_v)
            pltpu.sync_copy(emb.at[idx_v], rows_v)
            pltpu.sync_copy(rows_v, xpad.at[pl.ds(c * CH, CH)])


@_sc_kernel(
    out_type=(_f32(E, H), _f32(E, H), _f32(E, H)),
    scratch_types=[pltpu.VMEM((CH,), jnp.int32),
                   pltpu.VMEM((CH,), jnp.int32),
                   pltpu.VMEM((CH, H), jnp.float32)],
)
def _sc_gather_consts(zx, hx, rx, src2d, dst2d, xz_o, xh_o, xr_o,
                      idx_s, idx_d, rows_v):
    w = lax.axis_index("s") * NC + lax.axis_index("c")

    @pl.loop(0, -(-NCHUNK // NW))
    def _(j):
        c = w + NW * j

        @pl.when(c < NCHUNK)
        def _():
            pltpu.sync_copy(src2d.at[c], idx_s)
            pltpu.sync_copy(dst2d.at[c], idx_d)
            pltpu.sync_copy(zx.at[idx_s], rows_v)
            pltpu.sync_copy(rows_v, xz_o.at[pl.ds(c * CH, CH)])
            pltpu.sync_copy(hx.at[idx_s], rows_v)
            pltpu.sync_copy(rows_v, xh_o.at[pl.ds(c * CH, CH)])
            pltpu.sync_copy(rx.at[idx_d], rows_v)
            pltpu.sync_copy(rows_v, xr_o.at[pl.ds(c * CH, CH)])


@_sc_kernel(
    out_type=(_f32(E, H), _f32(E, H)),
    scratch_types=[pltpu.VMEM_SHARED((NT, H), jnp.float32),
                   pltpu.VMEM((CH,), jnp.int32),
                   pltpu.VMEM((CH, H), jnp.float32)],
)
def _sc_iter(m, rm, dst2d, src2d, zeros_n, g_m, g_rm, table, idx_v, rows_v):
    cid = lax.axis_index("c")
    sid = lax.axis_index("s")
    pltpu.sync_copy(zeros_n.at[pl.ds(sid * ZROWS, ZROWS)],
                    table.at[pl.ds(sid * ZROWS, ZROWS)])
    plsc.subcore_barrier()

    @pl.loop(0, ITER_J)
    def _(j):
        c = sid + NS * j

        @pl.when(c < NCHUNK)
        def _():
            pltpu.sync_copy(dst2d.at[c], idx_v)

            @pl.when(cid == 0)
            def _():
                pltpu.sync_copy(m.at[pl.ds(c * CH, CH)], rows_v)

            @pl.when(cid == 1)
            def _():
                pltpu.sync_copy(rm.at[pl.ds(c * CH, CH)], rows_v)

            pltpu.sync_copy(rows_v, table.at[idx_v], add=True)

    plsc.subcore_barrier()

    @pl.loop(0, ITER_J)
    def _(j):
        c = sid + NS * j

        @pl.when(c < NCHUNK)
        def _():
            pltpu.sync_copy(src2d.at[c], idx_v)
            pltpu.sync_copy(table.at[idx_v], rows_v)

            @pl.when(cid == 0)
            def _():
                pltpu.sync_copy(rows_v, g_m.at[pl.ds(c * CH, CH)])

            @pl.when(cid == 1)
            def _():
                pltpu.sync_copy(rows_v, g_rm.at[pl.ds(c * CH, CH)])


@_sc_kernel(
    out_type=(_f32(B, H), _f32(B, H), _f32(B, H)),
    scratch_types=[pltpu.VMEM_SHARED((NT, H), jnp.float32),
                   pltpu.VMEM((CH,), jnp.int32),
                   pltpu.VMEM((CH, H), jnp.float32)],
)
def _sc_final(m, dst2d, zeros_n, gx, root2d, nacc0_o, nacc1_o, gx_o,
              table, idx_v, rows_v):
    cid = lax.axis_index("c")
    sid = lax.axis_index("s")
    half = NCHUNK // 2
    pltpu.sync_copy(zeros_n.at[pl.ds(sid * ZROWS, ZROWS)],
                    table.at[pl.ds(sid * ZROWS, ZROWS)])
    plsc.subcore_barrier()

    @pl.loop(0, -(-half // NS))
    def _(j):
        cl = sid + NS * j

        @pl.when(cl < half)
        def _():
            c = cid * half + cl
            pltpu.sync_copy(dst2d.at[c], idx_v)
            pltpu.sync_copy(m.at[pl.ds(c * CH, CH)], rows_v)
            pltpu.sync_copy(rows_v, table.at[idx_v], add=True)

    plsc.subcore_barrier()

    @pl.when(sid < B // CH)
    def _():
        pltpu.sync_copy(root2d.at[sid], idx_v)
        pltpu.sync_copy(table.at[idx_v], rows_v)

        @pl.when(cid == 0)
        def _():
            pltpu.sync_copy(rows_v, nacc0_o.at[pl.ds(sid * CH, CH)])

        @pl.when(cid == 1)
        def _():
            pltpu.sync_copy(rows_v, nacc1_o.at[pl.ds(sid * CH, CH)])

    @pl.when((sid >= NS - B // CH) & (cid == 0))
    def _():
        s2 = sid - (NS - B // CH)
        pltpu.sync_copy(root2d.at[s2], idx_v)
        pltpu.sync_copy(gx.at[idx_v], rows_v)
        pltpu.sync_copy(rows_v, gx_o.at[pl.ds(s2 * CH, CH)])


def _tc_consts_body(x, wz, wh, wr, wg, bz, bh, br, bg, zx, hx, rx, gx):
    xb = x[...]
    zx[...] = xb @ wz[...] + bz[...]
    hx[...] = xb @ wh[...] + bh[...]
    rx[...] = xb @ wr[...] + br[...]
    gx[...] = xb @ wg[...] + bg[...]


def _tc_init_body(xz, xh, xr, au, m_o, rm_o):
    m = jax.nn.sigmoid(xz[...]) * jnp.tanh(xh[...])
    r = jax.nn.sigmoid(xr[...] + m @ au[...])
    m_o[...] = m
    rm_o[...] = r * m


def _tc_iter_body(gm, grm, mro, rmro, xz, xh, xr, az, ah, au, m_o, rm_o):
    s_acc = gm[...] - mro[...]
    rm_acc = grm[...] - rmro[...]
    z = jax.nn.sigmoid(xz[...] + s_acc @ az[...])
    mc = jnp.tanh(xh[...] + rm_acc @ ah[...])
    m = (1.0 - z) * s_acc + z * mc
    r = jax.nn.sigmoid(xr[...] + m @ au[...])
    m_o[...] = m
    rm_o[...] = r * m


def _tc_final_body(gxr, n0, n1, ag, out):
    out[...] = jax.nn.relu(gxr[...] + (n0[...] + n1[...]) @ ag[...])


def _wspec():
    return pl.BlockSpec((H, H), lambda i: (0, 0))


def _bspec():
    return pl.BlockSpec((1, H), lambda i: (0, 0))


def _espec(roll=False):
    if roll:
        return pl.BlockSpec((BE, H), lambda i: ((i + GE // 2) % GE, 0))
    return pl.BlockSpec((BE, H), lambda i: (i, 0))


_tc_consts = pl.pallas_call(
    _tc_consts_body,
    grid=(GN,),
    in_specs=[pl.BlockSpec((BN, H), lambda i: (i, 0))]
    + [_wspec()] * 4 + [_bspec()] * 4,
    out_specs=[pl.BlockSpec((BN, H), lambda i: (i, 0))] * 4,
    out_shape=[_f32(N, H)] * 4,
)

_tc_init = pl.pallas_call(
    _tc_init_body,
    grid=(GE,),
    in_specs=[_espec()] * 3 + [_wspec()],
    out_specs=[_espec()] * 2,
    out_shape=[_f32(E, H)] * 2,
)

_tc_iter = pl.pallas_call(
    _tc_iter_body,
    grid=(GE,),
    in_specs=[_espec(), _espec(), _espec(roll=True), _espec(roll=True),
              _espec(), _espec(), _espec()] + [_wspec()] * 3,
    out_specs=[_espec()] * 2,
    out_shape=[_f32(E, H)] * 2,
)

_tc_final = pl.pallas_call(
    _tc_final_body,
    grid=(1,),
    in_specs=[pl.BlockSpec((B, H), lambda i: (0, 0))] * 3 + [_wspec()],
    out_specs=pl.BlockSpec((B, H), lambda i: (0, 0)),
    out_shape=_f32(B, H),
)


def kernel(wid, edge_index, root_ids, embedding,
           W_z, b_z, W_r, U_r, b_ur, W_h, b_h, W_g, b_g):
    src = edge_index[0].astype(jnp.int32)
    dst = edge_index[1].astype(jnp.int32)
    wid2d = jnp.pad(wid.astype(jnp.int32), (0, NPAD - N)).reshape(XCH, CH)
    src2d = src.reshape(NCHUNK, CH)
    dst2d = dst.reshape(NCHUNK, CH)
    root2d = root_ids.astype(jnp.int32).reshape(B // CH, CH)
    zeros_n = jnp.zeros((NT, H), jnp.float32)

    wz1 = W_z[:, :H].T
    az = W_z[:, H:].T
    wh1 = W_h[:, :H].T
    ah = W_h[:, H:].T
    wr1 = W_r.T
    au = U_r.T
    wg1 = W_g[:, :H].T
    ag = W_g[:, H:].T
    bz = b_z.reshape(1, H)
    bh = b_h.reshape(1, H)
    br = b_ur.reshape(1, H)
    bg = b_g.reshape(1, H)

    xpad = _sc_gather_x(embedding, wid2d)
    x = xpad[:N]
    zx, hx, rx, gx = _tc_consts(x, wz1, wh1, wr1, wg1, bz, bh, br, bg)
    xz_src, xh_src, xr_dst = _sc_gather_consts(zx, hx, rx, src2d, dst2d)
    m, rm = _tc_init(xz_src, xh_src, xr_dst, au)
    for _ in range(5):
        g_m, g_rm = _sc_iter(m, rm, dst2d, src2d, zeros_n)
        m, rm = _tc_iter(g_m, g_rm, m, rm, xz_src, xh_src, xr_dst,
                         az, ah, au)
    nacc0, nacc1, gx_r = _sc_final(m, dst2d, zeros_n, gx, root2d)
    root_vecs = _tc_final(gx_r, nacc0, nacc1, ag)
    return (m, root_vecs)


# speedup vs baseline: 4.0381x; 1.4191x over previous
import functools
import jax
import jax.numpy as jnp
from jax import lax
from jax.experimental import pallas as pl
from jax.experimental.pallas import tpu as pltpu
from jax.experimental.pallas import tpu_sc as plsc

N = 10000
E = 160000
H = 128
NC = 2
NS = 16
NW = NC * NS
CH = 128
NCHUNK = E // CH
ITER_J = -(-NCHUNK // NS)
NT = 10240
ZROWS = NT // NS
NPAD = 10240
XCH = NPAD // CH
BE = 2000
GE = E // BE
BN = 1000
GN = N // BN
B = 256

def _f32(*shape):
    return jax.ShapeDtypeStruct(shape, jnp.float32)


def _sc_kernel(out_type, scratch_types):
    def deco(body):
        @functools.cache
        def build():
            mesh = plsc.VectorSubcoreMesh(
                core_axis_name="c", subcore_axis_name="s",
                num_cores=NC, num_subcores=NS)
            return pl.kernel(body, out_type=out_type, mesh=mesh,
                             scratch_types=scratch_types)

        def call(*args):
            return build()(*args)

        return call

    return deco


@_sc_kernel(
    out_type=_f32(NPAD, H),
    scratch_types=[pltpu.VMEM((CH,), jnp.int32),
                   pltpu.VMEM((CH, H), jnp.float32)],
)
def _sc_gather_x(emb, wid2d, xpad, idx_v, rows_v):
    w = lax.axis_index("s") * NC + lax.axis_index("c")

    @pl.loop(0, -(-XCH // NW))
    def _(j):
        c = w + NW * j

        @pl.when(c < XCH)
        def _():
            pltpu.sync_copy(wid2d.at[c], idx_v)
            pltpu.sync_copy(emb.at[idx_v], rows_v)
            pltpu.sync_copy(rows_v, xpad.at[pl.ds(c * CH, CH)])


_CONSTJ = -(-NCHUNK // NW)


@_sc_kernel(
    out_type=(_f32(E, H), _f32(E, H), _f32(E, H)),
    scratch_types=[pltpu.VMEM((2, CH), jnp.int32),
                   pltpu.VMEM((2, CH), jnp.int32),
                   pltpu.VMEM((3 * CH, H), jnp.float32),
                   pltpu.VMEM((3 * CH, H), jnp.float32),
                   pltpu.SemaphoreType.DMA,
                   pltpu.SemaphoreType.DMA,
                   pltpu.SemaphoreType.DMA,
                   pltpu.SemaphoreType.DMA],
)
def _sc_gather_consts(zx, hx, rx, src2d, dst2d, xz_o, xh_o, xr_o,
                      idx0, idx1, rows0, rows1, si0, si1, so0, so1):
    w = lax.axis_index("s") * NC + lax.axis_index("c")
    idx = (idx0, idx1)
    rows = (rows0, rows1)
    si = (si0, si1)
    so = (so0, so1)

    def chunk_of(k):
        return w + NW * k

    def issue_idx(k, b):
        c = chunk_of(k)

        @pl.when(c < NCHUNK)
        def _():
            pltpu.async_copy(src2d.at[c], idx[b].at[0], si[b])
            pltpu.async_copy(dst2d.at[c], idx[b].at[1], si[b])

    def outs(c):
        return ((zx, xz_o, 0), (hx, xh_o, 0), (rx, xr_o, 1))

    def step(k, t, b):
        c = chunk_of(k)

        @pl.when((t > 0) & (c - 2 * NW < NCHUNK))
        def _():
            for q in range(3):
                pltpu.make_async_copy(
                    rows[b].at[pl.ds(q * CH, CH)],
                    xz_o.at[pl.ds(0, CH)], so[b]).wait()

        @pl.when(c < NCHUNK)
        def _():
            pltpu.make_async_copy(src2d.at[c], idx[b].at[0], si[b]).wait()
            pltpu.make_async_copy(src2d.at[c], idx[b].at[1], si[b]).wait()

            for q, (tab, _, which) in enumerate(outs(c)):
                pltpu.sync_copy(tab.at[idx[b].at[which]],
                                rows[b].at[pl.ds(q * CH, CH)])
            for q, (_, out, _) in enumerate(outs(c)):
                pltpu.async_copy(rows[b].at[pl.ds(q * CH, CH)],
                                 out.at[pl.ds(c * CH, CH)], so[b])

        issue_idx(k + 2, b)

    issue_idx(0, 0)
    issue_idx(1, 1)

    @pl.loop(0, _CONSTJ // 2)
    def _(t):
        step(2 * t, t, 0)
        step(2 * t + 1, t, 1)

    for b in range(2):
        c = chunk_of(_CONSTJ - 2 + b)

        @pl.when(c < NCHUNK)
        def _():
            for q in range(3):
                pltpu.make_async_copy(rows[b].at[pl.ds(q * CH, CH)],
                                      xz_o.at[pl.ds(0, CH)], so[b]).wait()


SUP = 128
NSUP = E // SUP
SUPJ = ((-(-NSUP // NS) + 1) // 2) * 2
assert SUPJ % 2 == 0


@_sc_kernel(
    out_type=(_f32(E, H), _f32(E, H)),
    scratch_types=[pltpu.VMEM_SHARED((NT, H), jnp.float32),
                   pltpu.VMEM((1, CH), jnp.int32),
                   pltpu.VMEM((1, CH), jnp.int32),
                   pltpu.VMEM((SUP, H), jnp.float32),
                   pltpu.VMEM((SUP, H), jnp.float32),
                   pltpu.SemaphoreType.DMA,
                   pltpu.SemaphoreType.DMA,
                   pltpu.SemaphoreType.DMA,
                   pltpu.SemaphoreType.DMA,
                   pltpu.SemaphoreType.DMA,
                   pltpu.SemaphoreType.DMA],
)
def _sc_iter(m, rm, dst3d, src3d, zeros_n, g_m, g_rm,
             table, idx0, idx1, rows0, rows1, si0, si1, sr0, sr1, so0, so1):
    cid = lax.axis_index("c")
    sid = lax.axis_index("s")
    idx = (idx0, idx1)
    rows = (rows0, rows1)
    si = (si0, si1)
    sr = (sr0, sr1)
    so = (so0, so1)

    def sup_of(k):
        return sid + NS * k

    pltpu.sync_copy(zeros_n.at[pl.ds(sid * ZROWS, ZROWS)],
                    table.at[pl.ds(sid * ZROWS, ZROWS)])

    def issue_scatter_loads(k, b):
        s = sup_of(k)

        @pl.when(s < NSUP)
        def _():
            pltpu.async_copy(dst3d.at[s], idx[b], si[b])

            @pl.when(cid == 0)
            def _():
                pltpu.async_copy(m.at[pl.ds(s * SUP, SUP)], rows[b], sr[b])

            @pl.when(cid == 1)
            def _():
                pltpu.async_copy(rm.at[pl.ds(s * SUP, SUP)], rows[b], sr[b])

    def scatter_step(k, b):
        s = sup_of(k)

        @pl.when(s < NSUP)
        def _():
            pltpu.make_async_copy(dst3d.at[s], idx[b], si[b]).wait()
            pltpu.make_async_copy(m.at[pl.ds(s * SUP, SUP)],
                                  rows[b], sr[b]).wait()
            pltpu.sync_copy(rows[b], table.at[idx[b].at[0]], add=True)

        issue_scatter_loads(k + 2, b)

    issue_scatter_loads(0, 0)
    issue_scatter_loads(1, 1)
    plsc.subcore_barrier()

    @pl.loop(0, SUPJ // 2)
    def _(t):
        scatter_step(2 * t, 0)
        scatter_step(2 * t + 1, 1)

    plsc.subcore_barrier()

    def issue_gather_idx(k, b):
        s = sup_of(k)

        @pl.when(s < NSUP)
        def _():
            pltpu.async_copy(src3d.at[s], idx[b], si[b])

    def gather_step(k, t, b):
        s = sup_of(k)

        @pl.when((t > 0) & (s - 2 * NS < NSUP))
        def _():
            pltpu.make_async_copy(rows[b], g_m.at[pl.ds(0, SUP)],
                                  so[b]).wait()

        @pl.when(s < NSUP)
        def _():
            pltpu.make_async_copy(src3d.at[s], idx[b], si[b]).wait()
            pltpu.sync_copy(table.at[idx[b].at[0]], rows[b])

            @pl.when(cid == 0)
            def _():
                pltpu.async_copy(rows[b], g_m.at[pl.ds(s * SUP, SUP)], so[b])

            @pl.when(cid == 1)
            def _():
                pltpu.async_copy(rows[b], g_rm.at[pl.ds(s * SUP, SUP)], so[b])

        issue_gather_idx(k + 2, b)

    issue_gather_idx(0, 0)
    issue_gather_idx(1, 1)

    @pl.loop(0, SUPJ // 2)
    def _(t):
        gather_step(2 * t, t, 0)
        gather_step(2 * t + 1, t, 1)

    for b in range(2):
        s = sup_of(SUPJ - 2 + b)

        @pl.when(s < NSUP)
        def _():
            pltpu.make_async_copy(rows[b], g_m.at[pl.ds(0, SUP)],
                                  so[b]).wait()


@_sc_kernel(
    out_type=(_f32(B, H), _f32(B, H), _f32(B, H)),
    scratch_types=[pltpu.VMEM_SHARED((NT, H), jnp.float32),
                   pltpu.VMEM((CH,), jnp.int32),
                   pltpu.VMEM((CH, H), jnp.float32)],
)
def _sc_final(m, dst2d, zeros_n, gx, root2d, nacc0_o, nacc1_o, gx_o,
              table, idx_v, rows_v):
    cid = lax.axis_index("c")
    sid = lax.axis_index("s")
    half = NCHUNK // 2
    pltpu.sync_copy(zeros_n.at[pl.ds(sid * ZROWS, ZROWS)],
                    table.at[pl.ds(sid * ZROWS, ZROWS)])
    plsc.subcore_barrier()

    @pl.loop(0, -(-half // NS))
    def _(j):
        cl = sid + NS * j

        @pl.when(cl < half)
        def _():
            c = cid * half + cl
            pltpu.sync_copy(dst2d.at[c], idx_v)
            pltpu.sync_copy(m.at[pl.ds(c * CH, CH)], rows_v)
            pltpu.sync_copy(rows_v, table.at[idx_v], add=True)

    plsc.subcore_barrier()

    @pl.when(sid < B // CH)
    def _():
        pltpu.sync_copy(root2d.at[sid], idx_v)
        pltpu.sync_copy(table.at[idx_v], rows_v)

        @pl.when(cid == 0)
        def _():
            pltpu.sync_copy(rows_v, nacc0_o.at[pl.ds(sid * CH, CH)])

        @pl.when(cid == 1)
        def _():
            pltpu.sync_copy(rows_v, nacc1_o.at[pl.ds(sid * CH, CH)])

    @pl.when((sid >= NS - B // CH) & (cid == 0))
    def _():
        s2 = sid - (NS - B // CH)
        pltpu.sync_copy(root2d.at[s2], idx_v)
        pltpu.sync_copy(gx.at[idx_v], rows_v)
        pltpu.sync_copy(rows_v, gx_o.at[pl.ds(s2 * CH, CH)])


def _tc_consts_body(x, wz, wh, wr, wg, bz, bh, br, bg, zx, hx, rx, gx):
    xb = x[...]
    zx[...] = xb @ wz[...] + bz[...]
    hx[...] = xb @ wh[...] + bh[...]
    rx[...] = xb @ wr[...] + br[...]
    gx[...] = xb @ wg[...] + bg[...]


def _tc_init_body(xz, xh, xr, au, m_o, rm_o):
    m = jax.nn.sigmoid(xz[...]) * jnp.tanh(xh[...])
    r = jax.nn.sigmoid(xr[...] + m @ au[...])
    m_o[...] = m
    rm_o[...] = r * m


def _tc_iter_body(gm, grm, mro, rmro, xz, xh, xr, az, ah, au, m_o, rm_o):
    s_acc = gm[...] - mro[...]
    rm_acc = grm[...] - rmro[...]
    z = jax.nn.sigmoid(xz[...] + s_acc @ az[...])
    mc = jnp.tanh(xh[...] + rm_acc @ ah[...])
    m = (1.0 - z) * s_acc + z * mc
    r = jax.nn.sigmoid(xr[...] + m @ au[...])
    m_o[...] = m
    rm_o[...] = r * m


def _tc_final_body(gxr, n0, n1, ag, out):
    out[...] = jax.nn.relu(gxr[...] + (n0[...] + n1[...]) @ ag[...])


def _wspec():
    return pl.BlockSpec((H, H), lambda i: (0, 0))


def _bspec():
    return pl.BlockSpec((1, H), lambda i: (0, 0))


def _espec(roll=False):
    if roll:
        return pl.BlockSpec((BE, H), lambda i: ((i + GE // 2) % GE, 0))
    return pl.BlockSpec((BE, H), lambda i: (i, 0))


_tc_consts = pl.pallas_call(
    _tc_consts_body,
    grid=(GN,),
    in_specs=[pl.BlockSpec((BN, H), lambda i: (i, 0))]
    + [_wspec()] * 4 + [_bspec()] * 4,
    out_specs=[pl.BlockSpec((BN, H), lambda i: (i, 0))] * 4,
    out_shape=[_f32(N, H)] * 4,
)

_tc_init = pl.pallas_call(
    _tc_init_body,
    grid=(GE,),
    in_specs=[_espec()] * 3 + [_wspec()],
    out_specs=[_espec()] * 2,
    out_shape=[_f32(E, H)] * 2,
)

_tc_iter = pl.pallas_call(
    _tc_iter_body,
    grid=(GE,),
    in_specs=[_espec(), _espec(), _espec(roll=True), _espec(roll=True),
              _espec(), _espec(), _espec()] + [_wspec()] * 3,
    out_specs=[_espec()] * 2,
    out_shape=[_f32(E, H)] * 2,
)

_tc_final = pl.pallas_call(
    _tc_final_body,
    grid=(1,),
    in_specs=[pl.BlockSpec((B, H), lambda i: (0, 0))] * 3 + [_wspec()],
    out_specs=pl.BlockSpec((B, H), lambda i: (0, 0)),
    out_shape=_f32(B, H),
)


def kernel(wid, edge_index, root_ids, embedding,
           W_z, b_z, W_r, U_r, b_ur, W_h, b_h, W_g, b_g):
    src = edge_index[0].astype(jnp.int32)
    dst = edge_index[1].astype(jnp.int32)
    wid2d = jnp.pad(wid.astype(jnp.int32), (0, NPAD - N)).reshape(XCH, CH)
    src2d = src.reshape(NCHUNK, CH)
    dst2d = dst.reshape(NCHUNK, CH)
    src3d = src.reshape(NSUP, 1, CH)
    dst3d = dst.reshape(NSUP, 1, CH)
    root2d = root_ids.astype(jnp.int32).reshape(B // CH, CH)
    zeros_n = jnp.zeros((NT, H), jnp.float32)

    wz1 = W_z[:, :H].T
    az = W_z[:, H:].T
    wh1 = W_h[:, :H].T
    ah = W_h[:, H:].T
    wr1 = W_r.T
    au = U_r.T
    wg1 = W_g[:, :H].T
    ag = W_g[:, H:].T
    bz = b_z.reshape(1, H)
    bh = b_h.reshape(1, H)
    br = b_ur.reshape(1, H)
    bg = b_g.reshape(1, H)

    xpad = _sc_gather_x(embedding, wid2d)
    x = xpad[:N]
    zx, hx, rx, gx = _tc_consts(x, wz1, wh1, wr1, wg1, bz, bh, br, bg)
    xz_src, xh_src, xr_dst = _sc_gather_consts(zx, hx, rx, src2d, dst2d)
    m, rm = _tc_init(xz_src, xh_src, xr_dst, au)
    for _ in range(5):
        g_m, g_rm = _sc_iter(m, rm, dst3d, src3d, zeros_n)
        m, rm = _tc_iter(g_m, g_rm, m, rm, xz_src, xh_src, xr_dst,
                         az, ah, au)
    nacc0, nacc1, gx_r = _sc_final(m, dst2d, zeros_n, gx, root2d)
    root_vecs = _tc_final(gx_r, nacc0, nacc1, ag)
    return (m, root_vecs)


# speedup vs baseline: 4.3069x; 1.0666x over previous
import functools
import jax
import jax.numpy as jnp
from jax import lax
from jax.experimental import pallas as pl
from jax.experimental.pallas import tpu as pltpu
from jax.experimental.pallas import tpu_sc as plsc

N = 10000
E = 160000
H = 128
NC = 2
NS = 16
NW = NC * NS
CH = 128
NCHUNK = E // CH
ITER_J = -(-NCHUNK // NS)
NT = 10240
ZROWS = NT // NS
NPAD = 10240
XCH = NPAD // CH
BE = 2000
GE = E // BE
BN = 1000
GN = N // BN
B = 256

def _f32(*shape):
    return jax.ShapeDtypeStruct(shape, jnp.float32)


def _sc_kernel(out_type, scratch_types):
    def deco(body):
        @functools.cache
        def build():
            mesh = plsc.VectorSubcoreMesh(
                core_axis_name="c", subcore_axis_name="s",
                num_cores=NC, num_subcores=NS)
            return pl.kernel(body, out_type=out_type, mesh=mesh,
                             scratch_types=scratch_types)

        def call(*args):
            return build()(*args)

        return call

    return deco


@_sc_kernel(
    out_type=_f32(NPAD, H),
    scratch_types=[pltpu.VMEM((CH,), jnp.int32),
                   pltpu.VMEM((CH, H), jnp.float32)],
)
def _sc_gather_x(emb, wid2d, xpad, idx_v, rows_v):
    w = lax.axis_index("s") * NC + lax.axis_index("c")

    @pl.loop(0, -(-XCH // NW))
    def _(j):
        c = w + NW * j

        @pl.when(c < XCH)
        def _():
            pltpu.sync_copy(wid2d.at[c], idx_v)
            pltpu.sync_copy(emb.at[idx_v], rows_v)
            pltpu.sync_copy(rows_v, xpad.at[pl.ds(c * CH, CH)])


_CONSTJ = -(-NCHUNK // NW)


@_sc_kernel(
    out_type=(_f32(E, H), _f32(E, H), _f32(E, H)),
    scratch_types=[pltpu.VMEM((2, CH), jnp.int32),
                   pltpu.VMEM((2, CH), jnp.int32),
                   pltpu.VMEM((3 * CH, H), jnp.float32),
                   pltpu.VMEM((3 * CH, H), jnp.float32),
                   pltpu.SemaphoreType.DMA,
                   pltpu.SemaphoreType.DMA,
                   pltpu.SemaphoreType.DMA,
                   pltpu.SemaphoreType.DMA],
)
def _sc_gather_consts(zx, hx, rx, src2d, dst2d, xz_o, xh_o, xr_o,
                      idx0, idx1, rows0, rows1, si0, si1, so0, so1):
    w = lax.axis_index("s") * NC + lax.axis_index("c")
    idx = (idx0, idx1)
    rows = (rows0, rows1)
    si = (si0, si1)
    so = (so0, so1)

    def chunk_of(k):
        return w + NW * k

    def issue_idx(k, b):
        c = chunk_of(k)

        @pl.when(c < NCHUNK)
        def _():
            pltpu.async_copy(src2d.at[c], idx[b].at[0], si[b])
            pltpu.async_copy(dst2d.at[c], idx[b].at[1], si[b])

    def outs(c):
        return ((zx, xz_o, 0), (hx, xh_o, 0), (rx, xr_o, 1))

    def step(k, t, b):
        c = chunk_of(k)

        @pl.when((t > 0) & (c - 2 * NW < NCHUNK))
        def _():
            for q in range(3):
                pltpu.make_async_copy(
                    rows[b].at[pl.ds(q * CH, CH)],
                    xz_o.at[pl.ds(0, CH)], so[b]).wait()

        @pl.when(c < NCHUNK)
        def _():
            pltpu.make_async_copy(src2d.at[c], idx[b].at[0], si[b]).wait()
            pltpu.make_async_copy(src2d.at[c], idx[b].at[1], si[b]).wait()

            for q, (tab, _, which) in enumerate(outs(c)):
                pltpu.sync_copy(tab.at[idx[b].at[which]],
                                rows[b].at[pl.ds(q * CH, CH)])
            for q, (_, out, _) in enumerate(outs(c)):
                pltpu.async_copy(rows[b].at[pl.ds(q * CH, CH)],
                                 out.at[pl.ds(c * CH, CH)], so[b])

        issue_idx(k + 2, b)

    issue_idx(0, 0)
    issue_idx(1, 1)

    @pl.loop(0, _CONSTJ // 2)
    def _(t):
        step(2 * t, t, 0)
        step(2 * t + 1, t, 1)

    for b in range(2):
        c = chunk_of(_CONSTJ - 2 + b)

        @pl.when(c < NCHUNK)
        def _():
            for q in range(3):
                pltpu.make_async_copy(rows[b].at[pl.ds(q * CH, CH)],
                                      xz_o.at[pl.ds(0, CH)], so[b]).wait()


SUP = 80
NSUP = E // SUP
NBUF = 4
KJ = 128
assert KJ % NBUF == 0


@_sc_kernel(
    out_type=(_f32(E, H), _f32(E, H)),
    scratch_types=[pltpu.VMEM_SHARED((NT, H), jnp.float32)]
    + [pltpu.VMEM((1, SUP), jnp.int32)] * NBUF
    + [pltpu.VMEM((SUP, H), jnp.float32)] * NBUF
    + [pltpu.SemaphoreType.DMA] * (3 * NBUF),
)
def _sc_iter(m, rm, dst3d, src3d, zeros_n, g_m, g_rm, table, *bufs):
    idx = bufs[0:NBUF]
    rows = bufs[NBUF:2 * NBUF]
    si = bufs[2 * NBUF:3 * NBUF]
    sr = bufs[3 * NBUF:4 * NBUF]
    so = bufs[4 * NBUF:5 * NBUF]
    cid = lax.axis_index("c")
    sid = lax.axis_index("s")

    def sup_of(k):
        return sid + NS * k

    pltpu.sync_copy(zeros_n.at[pl.ds(sid * ZROWS, ZROWS)],
                    table.at[pl.ds(sid * ZROWS, ZROWS)])

    def issue_scatter_loads(k, b):
        s = sup_of(k)

        @pl.when(s < NSUP)
        def _():
            pltpu.async_copy(dst3d.at[s], idx[b], si[b])

            @pl.when(cid == 0)
            def _():
                pltpu.async_copy(m.at[pl.ds(s * SUP, SUP)], rows[b], sr[b])

            @pl.when(cid == 1)
            def _():
                pltpu.async_copy(rm.at[pl.ds(s * SUP, SUP)], rows[b], sr[b])

    def scatter_step(k, b):
        s = sup_of(k)

        @pl.when(s < NSUP)
        def _():
            pltpu.make_async_copy(dst3d.at[s], idx[b], si[b]).wait()
            pltpu.make_async_copy(m.at[pl.ds(s * SUP, SUP)],
                                  rows[b], sr[b]).wait()
            pltpu.sync_copy(rows[b], table.at[idx[b].at[0]], add=True)

        issue_scatter_loads(k + NBUF - 1, (b + NBUF - 1) % NBUF)

    for b in range(NBUF - 1):
        issue_scatter_loads(b, b)
    plsc.subcore_barrier()

    @pl.loop(0, KJ // NBUF)
    def _(t):
        for b in range(NBUF):
            scatter_step(NBUF * t + b, b)

    plsc.subcore_barrier()

    def issue_gather_idx(k, b):
        s = sup_of(k)

        @pl.when(s < NSUP)
        def _():
            pltpu.async_copy(src3d.at[s], idx[b], si[b])

    def gather_step(k, t, b):
        s = sup_of(k)

        @pl.when((t > 0) & (s - NBUF * NS < NSUP))
        def _():
            pltpu.make_async_copy(rows[b], g_m.at[pl.ds(0, SUP)],
                                  so[b]).wait()

        @pl.when(s < NSUP)
        def _():
            pltpu.make_async_copy(src3d.at[s], idx[b], si[b]).wait()
            pltpu.sync_copy(table.at[idx[b].at[0]], rows[b])

            @pl.when(cid == 0)
            def _():
                pltpu.async_copy(rows[b], g_m.at[pl.ds(s * SUP, SUP)], so[b])

            @pl.when(cid == 1)
            def _():
                pltpu.async_copy(rows[b], g_rm.at[pl.ds(s * SUP, SUP)], so[b])

        issue_gather_idx(k + NBUF - 1, (b + NBUF - 1) % NBUF)

    for b in range(NBUF - 1):
        issue_gather_idx(b, b)

    @pl.loop(0, KJ // NBUF)
    def _(t):
        for b in range(NBUF):
            gather_step(NBUF * t + b, t, b)

    for b in range(NBUF):
        s = sup_of(KJ - NBUF + b)

        @pl.when(s < NSUP)
        def _():
            pltpu.make_async_copy(rows[b], g_m.at[pl.ds(0, SUP)],
                                  so[b]).wait()


@_sc_kernel(
    out_type=(_f32(B, H), _f32(B, H), _f32(B, H)),
    scratch_types=[pltpu.VMEM_SHARED((NT, H), jnp.float32),
                   pltpu.VMEM((CH,), jnp.int32),
                   pltpu.VMEM((CH, H), jnp.float32)],
)
def _sc_final(m, dst2d, zeros_n, gx, root2d, nacc0_o, nacc1_o, gx_o,
              table, idx_v, rows_v):
    cid = lax.axis_index("c")
    sid = lax.axis_index("s")
    half = NCHUNK // 2
    pltpu.sync_copy(zeros_n.at[pl.ds(sid * ZROWS, ZROWS)],
                    table.at[pl.ds(sid * ZROWS, ZROWS)])
    plsc.subcore_barrier()

    @pl.loop(0, -(-half // NS))
    def _(j):
        cl = sid + NS * j

        @pl.when(cl < half)
        def _():
            c = cid * half + cl
            pltpu.sync_copy(dst2d.at[c], idx_v)
            pltpu.sync_copy(m.at[pl.ds(c * CH, CH)], rows_v)
            pltpu.sync_copy(rows_v, table.at[idx_v], add=True)

    plsc.subcore_barrier()

    @pl.when(sid < B // CH)
    def _():
        pltpu.sync_copy(root2d.at[sid], idx_v)
        pltpu.sync_copy(table.at[idx_v], rows_v)

        @pl.when(cid == 0)
        def _():
            pltpu.sync_copy(rows_v, nacc0_o.at[pl.ds(sid * CH, CH)])

        @pl.when(cid == 1)
        def _():
            pltpu.sync_copy(rows_v, nacc1_o.at[pl.ds(sid * CH, CH)])

    @pl.when((sid >= NS - B // CH) & (cid == 0))
    def _():
        s2 = sid - (NS - B // CH)
        pltpu.sync_copy(root2d.at[s2], idx_v)
        pltpu.sync_copy(gx.at[idx_v], rows_v)
        pltpu.sync_copy(rows_v, gx_o.at[pl.ds(s2 * CH, CH)])


def _tc_consts_body(x, wz, wh, wr, wg, bz, bh, br, bg, zx, hx, rx, gx):
    xb = x[...]
    zx[...] = xb @ wz[...] + bz[...]
    hx[...] = xb @ wh[...] + bh[...]
    rx[...] = xb @ wr[...] + br[...]
    gx[...] = xb @ wg[...] + bg[...]


def _tc_init_body(xz, xh, xr, au, m_o, rm_o, xzb_o, xhb_o, xrb_o):
    m = jax.nn.sigmoid(xz[...]) * jnp.tanh(xh[...])
    r = jax.nn.sigmoid(xr[...] + m @ au[...])
    m_o[...] = m
    rm_o[...] = r * m
    xzb_o[...] = xz[...].astype(jnp.bfloat16)
    xhb_o[...] = xh[...].astype(jnp.bfloat16)
    xrb_o[...] = xr[...].astype(jnp.bfloat16)


def _tc_iter_body(gm, grm, mro, rmro, xz, xh, xr, az, ah, au, m_o, rm_o):
    s_acc = gm[...] - mro[...]
    rm_acc = grm[...] - rmro[...]
    z = jax.nn.sigmoid(xz[...].astype(jnp.float32) + s_acc @ az[...])
    mc = jnp.tanh(xh[...].astype(jnp.float32) + rm_acc @ ah[...])
    m = (1.0 - z) * s_acc + z * mc
    r = jax.nn.sigmoid(xr[...].astype(jnp.float32) + m @ au[...])
    m_o[...] = m
    rm_o[...] = r * m


def _tc_final_body(gxr, n0, n1, ag, out):
    out[...] = jax.nn.relu(gxr[...] + (n0[...] + n1[...]) @ ag[...])


def _wspec():
    return pl.BlockSpec((H, H), lambda i: (0, 0))


def _bspec():
    return pl.BlockSpec((1, H), lambda i: (0, 0))


def _espec(roll=False):
    if roll:
        return pl.BlockSpec((BE, H), lambda i: ((i + GE // 2) % GE, 0))
    return pl.BlockSpec((BE, H), lambda i: (i, 0))


_tc_consts = pl.pallas_call(
    _tc_consts_body,
    grid=(GN,),
    in_specs=[pl.BlockSpec((BN, H), lambda i: (i, 0))]
    + [_wspec()] * 4 + [_bspec()] * 4,
    out_specs=[pl.BlockSpec((BN, H), lambda i: (i, 0))] * 4,
    out_shape=[_f32(N, H)] * 4,
)

_tc_init = pl.pallas_call(
    _tc_init_body,
    grid=(GE,),
    in_specs=[_espec()] * 3 + [_wspec()],
    out_specs=[_espec()] * 5,
    out_shape=[_f32(E, H)] * 2
    + [jax.ShapeDtypeStruct((E, H), jnp.bfloat16)] * 3,
)

_tc_iter = pl.pallas_call(
    _tc_iter_body,
    grid=(GE,),
    in_specs=[_espec(), _espec(), _espec(roll=True), _espec(roll=True),
              _espec(), _espec(), _espec()] + [_wspec()] * 3,
    out_specs=[_espec()] * 2,
    out_shape=[_f32(E, H)] * 2,
)

_tc_final = pl.pallas_call(
    _tc_final_body,
    grid=(1,),
    in_specs=[pl.BlockSpec((B, H), lambda i: (0, 0))] * 3 + [_wspec()],
    out_specs=pl.BlockSpec((B, H), lambda i: (0, 0)),
    out_shape=_f32(B, H),
)


def kernel(wid, edge_index, root_ids, embedding,
           W_z, b_z, W_r, U_r, b_ur, W_h, b_h, W_g, b_g):
    src = edge_index[0].astype(jnp.int32)
    dst = edge_index[1].astype(jnp.int32)
    wid2d = jnp.pad(wid.astype(jnp.int32), (0, NPAD - N)).reshape(XCH, CH)
    src2d = src.reshape(NCHUNK, CH)
    dst2d = dst.reshape(NCHUNK, CH)
    src3d = src.reshape(NSUP, 1, SUP)
    dst3d = dst.reshape(NSUP, 1, SUP)
    root2d = root_ids.astype(jnp.int32).reshape(B // CH, CH)
    zeros_n = jnp.zeros((NT, H), jnp.float32)

    wz1 = W_z[:, :H].T
    az = W_z[:, H:].T
    wh1 = W_h[:, :H].T
    ah = W_h[:, H:].T
    wr1 = W_r.T
    au = U_r.T
    wg1 = W_g[:, :H].T
    ag = W_g[:, H:].T
    bz = b_z.reshape(1, H)
    bh = b_h.reshape(1, H)
    br = b_ur.reshape(1, H)
    bg = b_g.reshape(1, H)

    xpad = _sc_gather_x(embedding, wid2d)
    x = xpad[:N]
    zx, hx, rx, gx = _tc_consts(x, wz1, wh1, wr1, wg1, bz, bh, br, bg)
    xz_src, xh_src, xr_dst = _sc_gather_consts(zx, hx, rx, src2d, dst2d)
    m, rm, xzb, xhb, xrb = _tc_init(xz_src, xh_src, xr_dst, au)
    for _ in range(5):
        g_m, g_rm = _sc_iter(m, rm, dst3d, src3d, zeros_n)
        m, rm = _tc_iter(g_m, g_rm, m, rm, xzb, xhb, xrb, az, ah, au)
    nacc0, nacc1, gx_r = _sc_final(m, dst2d, zeros_n, gx, root2d)
    root_vecs = _tc_final(gx_r, nacc0, nacc1, ag)
    return (m, root_vecs)
